# baseline (speedup 1.0000x reference)
import functools

import jax
import jax.numpy as jnp
from jax import lax
from jax.experimental import pallas as pl
from jax.experimental.pallas import tpu as pltpu
from jax.experimental.pallas import tpu_sc as plsc

N = 10000
D = 128
H = 64
E = 320000
NGEN = 1024

NC = 2
NS = 16
NW = NC * NS
NPAD = 10112
RPT = NPAD // NS
EPAD = 327680
EPT = EPAD // NW
CH = 512
NCH = EPT // CH

_mesh = plsc.VectorSubcoreMesh(core_axis_name="c", subcore_axis_name="s")
_sc_params = pltpu.CompilerParams(use_tc_tiling_on_sc=False)


@functools.partial(
    pl.kernel,
    out_type=jax.ShapeDtypeStruct((NC * NPAD, H), jnp.float32),
    mesh=_mesh,
    compiler_params=_sc_params,
    scratch_types=[
        pltpu.VMEM_SHARED((NPAD, H), jnp.float32),
        pltpu.VMEM((EPT,), jnp.int32),
        pltpu.VMEM((EPT,), jnp.int32),
        pltpu.VMEM((2, CH, H), jnp.float32),
        pltpu.SemaphoreType.DMA,
        pltpu.SemaphoreType.DMA,
    ],
)
def _edge_pass64(table, src1, dst1, zrows, out, acc, sidx, didx, rbuf,
                 sem0, sem1):
  c = lax.axis_index("c")
  s = lax.axis_index("s")
  wid = c * NS + s
  pltpu.sync_copy(zrows, acc.at[pl.ds(s * RPT, RPT)])
  pltpu.sync_copy(src1.at[pl.ds(wid * EPT, EPT)], sidx)
  pltpu.sync_copy(dst1.at[pl.ds(wid * EPT, EPT)], didx)
  plsc.subcore_barrier()

  sems = (sem0, sem1)

  def fire(g, b):
    pltpu.async_copy(table.at[sidx.at[pl.ds(g * CH, CH)]], rbuf.at[b],
                     sems[b])

  def wait_scatter(g, b):
    pltpu.make_async_copy(table.at[sidx.at[pl.ds(g * CH, CH)]], rbuf.at[b],
                          sems[b]).wait()
    pltpu.sync_copy(rbuf.at[b], acc.at[didx.at[pl.ds(g * CH, CH)]], add=True)

  fire(0, 0)
  fire(1, 1)

  def body(p, carry):
    g0 = 2 * p
    wait_scatter(g0, 0)
    fire(g0 + 2, 0)
    wait_scatter(g0 + 1, 1)
    fire(g0 + 3, 1)
    return carry

  lax.fori_loop(0, NCH // 2 - 1, body, 0)
  wait_scatter(NCH - 2, 0)
  wait_scatter(NCH - 1, 1)

  plsc.subcore_barrier()
  pltpu.sync_copy(acc.at[pl.ds(s * RPT, RPT)],
                  out.at[pl.ds(c * NPAD + s * RPT, RPT)])


L2E = 2 * EPT
L2H = L2E // 2


@functools.partial(
    pl.kernel,
    out_type=jax.ShapeDtypeStruct((NC * 2 * NPAD,), jnp.float32),
    mesh=_mesh,
    compiler_params=_sc_params,
    scratch_types=[
        pltpu.VMEM_SHARED((2 * NPAD,), jnp.float32),
        pltpu.VMEM((L2E,), jnp.int32),
        pltpu.VMEM((L2E,), jnp.int32),
        pltpu.VMEM((2, L2H), jnp.float32),
        pltpu.SemaphoreType.DMA,
        pltpu.SemaphoreType.DMA,
    ],
)
def _edge_pass2(table, src1, dst1, zrows, out, acc, sidx, didx, rbuf,
                sem0, sem1):
  zlen = 2 * NPAD // NS
  c = lax.axis_index("c")
  s = lax.axis_index("s")
  wid = c * NS + s
  pltpu.sync_copy(zrows, acc.at[pl.ds(s * zlen, zlen)])
  pltpu.sync_copy(src1.at[pl.ds(wid * L2E, L2E)], sidx)
  pltpu.sync_copy(dst1.at[pl.ds(wid * L2E, L2E)], didx)
  plsc.subcore_barrier()

  sems = (sem0, sem1)
  for b in (0, 1):
    pltpu.async_copy(table.at[sidx.at[pl.ds(b * L2H, L2H)]], rbuf.at[b],
                     sems[b])
  for b in (0, 1):
    pltpu.make_async_copy(table.at[sidx.at[pl.ds(b * L2H, L2H)]], rbuf.at[b],
                          sems[b]).wait()
    pltpu.sync_copy(rbuf.at[b], acc.at[didx.at[pl.ds(b * L2H, L2H)]],
                    add=True)

  plsc.subcore_barrier()
  pltpu.sync_copy(acc.at[pl.ds(s * zlen, zlen)],
                  out.at[pl.ds(c * 2 * NPAD + s * zlen, zlen)])


@functools.partial(
    pl.kernel,
    out_type=jax.ShapeDtypeStruct((NC * NPAD,), jnp.float32),
    mesh=_mesh,
    compiler_params=_sc_params,
    scratch_types=[
        pltpu.VMEM_SHARED((NPAD,), jnp.float32),
        pltpu.VMEM((EPT,), jnp.int32),
        pltpu.VMEM((EPT,), jnp.float32),
    ],
)
def _hist_kernel(dst1, zrows, ones_h, out, acc, didx, onesv):
  c = lax.axis_index("c")
  s = lax.axis_index("s")
  wid = c * NS + s
  pltpu.sync_copy(zrows, acc.at[pl.ds(s * RPT, RPT)])
  pltpu.sync_copy(dst1.at[pl.ds(wid * EPT, EPT)], didx)
  pltpu.sync_copy(ones_h, onesv)
  plsc.subcore_barrier()
  pltpu.sync_copy(onesv, acc.at[didx], add=True)
  plsc.subcore_barrier()
  pltpu.sync_copy(acc.at[pl.ds(s * RPT, RPT)],
                  out.at[pl.ds(c * NPAD + s * RPT, RPT)])


def _dense1_body(x_ref, w1_ref, hp_ref, hs_ref, dis_ref):
  deg = hp_ref[0:NPAD] + hp_ref[NPAD:2 * NPAD] + 1.0
  dis = lax.rsqrt(deg)
  h = jnp.dot(x_ref[...], w1_ref[...], preferred_element_type=jnp.float32)
  dis_ref[...] = dis
  hs_ref[0:N] = h * dis[0:N]


def _dense2_body(accs_ref, hs_ref, dis_ref, b1_ref, w2_ref, zs_ref):
  acc = accs_ref[0:N] + accs_ref[NPAD:NPAD + N]
  dis = dis_ref[0:N]
  out1 = jnp.maximum(dis * (acc + hs_ref[0:N]) + b1_ref[...], 0.0)
  z = jnp.dot(out1, w2_ref[...], preferred_element_type=jnp.float32)
  zs_ref[0:N] = (dis * z)[:, 0:2]


def _final_body(acc2_ref, zs_ref, dis_ref, b2_ref, out_ref):
  a = acc2_ref[0:NGEN] + acc2_ref[NPAD:NPAD + NGEN]
  out_ref[...] = dis_ref[0:NGEN] * (a + zs_ref[0:NGEN]) + b2_ref[...]


def kernel(x, edge_index, W1, b1, W2, b2):
  src = edge_index[0].astype(jnp.int32)
  dst = edge_index[1].astype(jnp.int32)
  padi = N + jnp.arange(EPAD - E, dtype=jnp.int32) % (NPAD - N)
  srcp = jnp.concatenate([src, padi])
  dstp = jnp.concatenate([dst, padi])
  sint = (2 * srcp[:, None] + jnp.arange(2, dtype=jnp.int32)[None, :]).reshape(-1)
  dint = (2 * dstp[:, None] + jnp.arange(2, dtype=jnp.int32)[None, :]).reshape(-1)

  z64 = jnp.zeros((RPT, H), jnp.float32)
  z1 = jnp.zeros((RPT,), jnp.float32)
  z2 = jnp.zeros((2 * NPAD // NS,), jnp.float32)
  ones1 = jnp.ones((EPT,), jnp.float32)

  hp1 = _hist_kernel(dstp, z1, ones1)
  hp = hp1.reshape(NC * NPAD, 1)

  hs, dis = pl.pallas_call(
      _dense1_body,
      out_shape=(jax.ShapeDtypeStruct((NPAD, H), jnp.float32),
                 jax.ShapeDtypeStruct((NPAD, 1), jnp.float32)),
  )(x, W1, hp)

  accs = _edge_pass64(hs, srcp, dstp, z64)

  W2p = jnp.zeros((H, 8), jnp.float32).at[:, 0:2].set(W2)
  zs = pl.pallas_call(
      _dense2_body,
      out_shape=jax.ShapeDtypeStruct((NPAD, 2), jnp.float32),
  )(accs, hs, dis, b1, W2p)

  acc2f = _edge_pass2(zs.reshape(-1), sint, dint, z2)
  acc2 = acc2f.reshape(NC * NPAD, 2)

  out = pl.pallas_call(
      _final_body,
      out_shape=jax.ShapeDtypeStruct((NGEN, 2), jnp.float32),
  )(acc2, zs, dis, b2)

  return out.reshape(-1)


# speedup vs baseline: 2.8236x; 2.8236x over previous
import functools

import jax
import jax.numpy as jnp
from jax import lax
from jax.experimental import pallas as pl
from jax.experimental.pallas import tpu as pltpu
from jax.experimental.pallas import tpu_sc as plsc

N = 10000
D = 128
H = 64
E = 320000
NGEN = 1024

NC = 2
NS = 16
NW = NC * NS
NPAD = 10112
RPT = NPAD // NS
EPAD = 327680
EPT = EPAD // NW
CH = 128
K = 4
GRP = K * CH
NG = EPT // GRP

_mesh = plsc.VectorSubcoreMesh(core_axis_name="c", subcore_axis_name="s")
_sc_params = pltpu.CompilerParams(use_tc_tiling_on_sc=False)


@functools.partial(
    pl.kernel,
    out_type=jax.ShapeDtypeStruct((NC * NPAD, H), jnp.float32),
    mesh=_mesh,
    compiler_params=_sc_params,
    scratch_types=[
        pltpu.VMEM_SHARED((NPAD, H), jnp.float32),
        pltpu.VMEM((EPT,), jnp.int32),
        pltpu.VMEM((EPT,), jnp.int32),
        pltpu.VMEM((2 * K, CH, H), jnp.float32),
        pltpu.SemaphoreType.DMA,
        pltpu.SemaphoreType.DMA,
    ],
)
def _edge_pass64(table, src1, dst1, zrows, out, acc, sidx, didx, rbuf,
                 sem0, sem1):
  c = lax.axis_index("c")
  s = lax.axis_index("s")
  wid = c * NS + s
  pltpu.sync_copy(zrows, acc.at[pl.ds(s * RPT, RPT)])
  pltpu.sync_copy(src1.at[pl.ds(wid * EPT, EPT)], sidx)
  pltpu.sync_copy(dst1.at[pl.ds(wid * EPT, EPT)], didx)
  plsc.subcore_barrier()

  sems = (sem0, sem1)

  def fire(g, b):
    for i in range(K):
      pltpu.async_copy(table.at[sidx.at[pl.ds(g * GRP + i * CH, CH)]],
                       rbuf.at[b * K + i], sems[b])

  def wait_scatter(g, b):
    for i in range(K):
      pltpu.make_async_copy(table.at[sidx.at[pl.ds(g * GRP + i * CH, CH)]],
                            rbuf.at[b * K + i], sems[b]).wait()
      pltpu.sync_copy(rbuf.at[b * K + i],
                      acc.at[didx.at[pl.ds(g * GRP + i * CH, CH)]], add=True)

  fire(0, 0)
  fire(1, 1)

  def body(p, carry):
    g0 = 2 * p
    wait_scatter(g0, 0)
    fire(g0 + 2, 0)
    wait_scatter(g0 + 1, 1)
    fire(g0 + 3, 1)
    return carry

  lax.fori_loop(0, NG // 2 - 1, body, 0)
  wait_scatter(NG - 2, 0)
  wait_scatter(NG - 1, 1)

  plsc.subcore_barrier()
  pltpu.sync_copy(acc.at[pl.ds(s * RPT, RPT)],
                  out.at[pl.ds(c * NPAD + s * RPT, RPT)])


L2E = 2 * EPT
NG2 = L2E // GRP


@functools.partial(
    pl.kernel,
    out_type=jax.ShapeDtypeStruct((NC * 2 * NPAD,), jnp.float32),
    mesh=_mesh,
    compiler_params=_sc_params,
    scratch_types=[
        pltpu.VMEM_SHARED((2 * NPAD,), jnp.float32),
        pltpu.VMEM((L2E,), jnp.int32),
        pltpu.VMEM((L2E,), jnp.int32),
        pltpu.VMEM((2 * K, CH), jnp.float32),
        pltpu.SemaphoreType.DMA,
        pltpu.SemaphoreType.DMA,
    ],
)
def _edge_pass2(table, src1, dst1, zrows, out, acc, sidx, didx, rbuf,
                sem0, sem1):
  zlen = 2 * NPAD // NS
  c = lax.axis_index("c")
  s = lax.axis_index("s")
  wid = c * NS + s
  pltpu.sync_copy(zrows, acc.at[pl.ds(s * zlen, zlen)])
  pltpu.sync_copy(src1.at[pl.ds(wid * L2E, L2E)], sidx)
  pltpu.sync_copy(dst1.at[pl.ds(wid * L2E, L2E)], didx)
  plsc.subcore_barrier()

  sems = (sem0, sem1)

  def fire(g, b):
    for i in range(K):
      pltpu.async_copy(table.at[sidx.at[pl.ds(g * GRP + i * CH, CH)]],
                       rbuf.at[b * K + i], sems[b])

  def wait_scatter(g, b):
    for i in range(K):
      pltpu.make_async_copy(table.at[sidx.at[pl.ds(g * GRP + i * CH, CH)]],
                            rbuf.at[b * K + i], sems[b]).wait()
      pltpu.sync_copy(rbuf.at[b * K + i],
                      acc.at[didx.at[pl.ds(g * GRP + i * CH, CH)]], add=True)

  fire(0, 0)
  fire(1, 1)

  def body(p, carry):
    g0 = 2 * p
    wait_scatter(g0, 0)
    fire(g0 + 2, 0)
    wait_scatter(g0 + 1, 1)
    fire(g0 + 3, 1)
    return carry

  lax.fori_loop(0, NG2 // 2 - 1, body, 0)
  wait_scatter(NG2 - 2, 0)
  wait_scatter(NG2 - 1, 1)

  plsc.subcore_barrier()
  pltpu.sync_copy(acc.at[pl.ds(s * zlen, zlen)],
                  out.at[pl.ds(c * 2 * NPAD + s * zlen, zlen)])


@functools.partial(
    pl.kernel,
    out_type=jax.ShapeDtypeStruct((NC * NPAD,), jnp.float32),
    mesh=_mesh,
    compiler_params=_sc_params,
    scratch_types=[
        pltpu.VMEM_SHARED((NPAD,), jnp.float32),
        pltpu.VMEM((EPT,), jnp.int32),
        pltpu.VMEM((EPT,), jnp.float32),
    ],
)
def _hist_kernel(dst1, zrows, ones_h, out, acc, didx, onesv):
  c = lax.axis_index("c")
  s = lax.axis_index("s")
  wid = c * NS + s
  pltpu.sync_copy(zrows, acc.at[pl.ds(s * RPT, RPT)])
  pltpu.sync_copy(dst1.at[pl.ds(wid * EPT, EPT)], didx)
  pltpu.sync_copy(ones_h, onesv)
  plsc.subcore_barrier()
  pltpu.sync_copy(onesv, acc.at[didx], add=True)
  plsc.subcore_barrier()
  pltpu.sync_copy(acc.at[pl.ds(s * RPT, RPT)],
                  out.at[pl.ds(c * NPAD + s * RPT, RPT)])


def _mm_body(x_ref, w1_ref, h_ref):
  h_ref[...] = jnp.dot(x_ref[...], w1_ref[...],
                       preferred_element_type=jnp.float32)


def _dense1_body(hp_ref, h_ref, hs_ref, dis_ref):
  deg = hp_ref[0:NPAD] + hp_ref[NPAD:2 * NPAD] + 1.0
  dis = lax.rsqrt(deg)
  dis_ref[...] = dis
  hs_ref[0:N] = h_ref[...] * dis[0:N][:, None]


def _dense2_body(accs_ref, hs_ref, dis_ref, b1_ref, w2_ref, zs_ref):
  acc = accs_ref[0:N] + accs_ref[NPAD:NPAD + N]
  dis = dis_ref[0:N][:, None]
  out1 = jnp.maximum(dis * (acc + hs_ref[0:N]) + b1_ref[...], 0.0)
  z = jnp.dot(out1, w2_ref[...], preferred_element_type=jnp.float32)
  zs_ref[0:N] = (dis * z)[:, 0:2]


def _final_body(acc2_ref, zf_ref, di_ref, b2i_ref, out_ref):
  a = acc2_ref[0:2 * NGEN] + acc2_ref[2 * NPAD:2 * NPAD + 2 * NGEN]
  out_ref[...] = di_ref[...] * (a + zf_ref[0:2 * NGEN]) + b2i_ref[...]


def kernel(x, edge_index, W1, b1, W2, b2):
  src = edge_index[0].astype(jnp.int32)
  dst = edge_index[1].astype(jnp.int32)
  padi = N + jnp.arange(EPAD - E, dtype=jnp.int32) % (NPAD - N)
  srcp = jnp.concatenate([src, padi])
  dstp = jnp.concatenate([dst, padi])
  sw = srcp.reshape(NW, EPT)
  dw = dstp.reshape(NW, EPT)
  sint = jnp.concatenate([2 * sw, 2 * sw + 1], axis=1).reshape(-1)
  dint = jnp.concatenate([2 * dw, 2 * dw + 1], axis=1).reshape(-1)

  z64 = jnp.zeros((RPT, H), jnp.float32)
  z1 = jnp.zeros((RPT,), jnp.float32)
  z2 = jnp.zeros((2 * NPAD // NS,), jnp.float32)
  ones1 = jnp.ones((EPT,), jnp.float32)

  h = pl.pallas_call(
      _mm_body, out_shape=jax.ShapeDtypeStruct((N, H), jnp.float32),
  )(x, W1)

  hp1 = _hist_kernel(dstp, z1, ones1)

  hs, dis1 = pl.pallas_call(
      _dense1_body,
      out_shape=(jax.ShapeDtypeStruct((NPAD, H), jnp.float32),
                 jax.ShapeDtypeStruct((NPAD,), jnp.float32)),
  )(hp1, h)

  accs = _edge_pass64(hs, srcp, dstp, z64)

  W2p = jnp.zeros((H, 8), jnp.float32).at[:, 0:2].set(W2)
  zs = pl.pallas_call(
      _dense2_body,
      out_shape=jax.ShapeDtypeStruct((NPAD, 2), jnp.float32),
  )(accs, hs, dis1, b1, W2p)

  zsf = zs.reshape(-1)
  acc2f = _edge_pass2(zsf, sint, dint, z2)

  di = jnp.repeat(dis1[0:NGEN], 2)
  b2i = jnp.tile(b2, NGEN)
  out = pl.pallas_call(
      _final_body,
      out_shape=jax.ShapeDtypeStruct((2 * NGEN,), jnp.float32),
  )(acc2f, zsf, di, b2i)

  return out


# speedup vs baseline: 3.5818x; 1.2685x over previous
import functools

import jax
import jax.numpy as jnp
from jax import lax
from jax.experimental import pallas as pl
from jax.experimental.pallas import tpu as pltpu
from jax.experimental.pallas import tpu_sc as plsc

N = 10000
D = 128
H = 64
E = 320000
NGEN = 1024

NC = 2
NS = 16
NW = NC * NS
NPAD = 10112
RPT = NPAD // NS
EPAD = 327680
EPT = EPAD // NW
CH = 128
K = 4
GRP = K * CH
NG = EPT // GRP

_mesh = plsc.VectorSubcoreMesh(core_axis_name="c", subcore_axis_name="s")
_sc_params = pltpu.CompilerParams(use_tc_tiling_on_sc=False)


@functools.partial(
    pl.kernel,
    out_type=jax.ShapeDtypeStruct((NC * NPAD, H), jnp.float32),
    mesh=_mesh,
    compiler_params=_sc_params,
    scratch_types=[
        pltpu.VMEM_SHARED((NPAD, H), jnp.float32),
        pltpu.VMEM((EPT,), jnp.int32),
        pltpu.VMEM((EPT,), jnp.int32),
        pltpu.VMEM((2 * K, CH, H), jnp.float32),
        pltpu.SemaphoreType.DMA,
        pltpu.SemaphoreType.DMA,
    ],
)
def _edge_pass64(table, src1, dst1, zrows, out, acc, sidx, didx, rbuf,
                 sem0, sem1):
  c = lax.axis_index("c")
  s = lax.axis_index("s")
  wid = c * NS + s
  pltpu.sync_copy(zrows, acc.at[pl.ds(s * RPT, RPT)])
  pltpu.sync_copy(src1.at[pl.ds(wid * EPT, EPT)], sidx)
  pltpu.sync_copy(dst1.at[pl.ds(wid * EPT, EPT)], didx)
  plsc.subcore_barrier()

  sems = (sem0, sem1)

  def fire(g, b):
    for i in range(K):
      pltpu.async_copy(table.at[sidx.at[pl.ds(g * GRP + i * CH, CH)]],
                       rbuf.at[b * K + i], sems[b])

  def wait_scatter(g, b):
    for i in range(K):
      pltpu.make_async_copy(table.at[sidx.at[pl.ds(g * GRP + i * CH, CH)]],
                            rbuf.at[b * K + i], sems[b]).wait()
      pltpu.sync_copy(rbuf.at[b * K + i],
                      acc.at[didx.at[pl.ds(g * GRP + i * CH, CH)]], add=True)

  fire(0, 0)
  fire(1, 1)

  def body(p, carry):
    g0 = 2 * p
    wait_scatter(g0, 0)
    fire(g0 + 2, 0)
    wait_scatter(g0 + 1, 1)
    fire(g0 + 3, 1)
    return carry

  lax.fori_loop(0, NG // 2 - 1, body, 0)
  wait_scatter(NG - 2, 0)
  wait_scatter(NG - 1, 1)

  plsc.subcore_barrier()
  pltpu.sync_copy(acc.at[pl.ds(s * RPT, RPT)],
                  out.at[pl.ds(c * NPAD + s * RPT, RPT)])


ZL = 2 * NPAD
CSL = ZL // NS


@functools.partial(
    pl.kernel,
    out_type=jax.ShapeDtypeStruct((NC * ZL,), jnp.float32),
    mesh=_mesh,
    compiler_params=pltpu.CompilerParams(use_tc_tiling_on_sc=False,
                                         needs_layout_passes=False),
    scratch_types=[
        pltpu.VMEM_SHARED((NS, ZL), jnp.float32),
        pltpu.VMEM((ZL,), jnp.float32),
        pltpu.VMEM((ZL,), jnp.float32),
        pltpu.VMEM((EPT,), jnp.int32),
        pltpu.VMEM((EPT,), jnp.int32),
        pltpu.VMEM((CSL,), jnp.float32),
        pltpu.VMEM((CSL,), jnp.float32),
    ],
)
def _edge_pass2(zsf, src1, dst1, zl_zero, out, stage, tab, acc, sidx, didx,
                cin, csum):
  c = lax.axis_index("c")
  s = lax.axis_index("s")
  wid = c * NS + s
  pltpu.sync_copy(zsf, tab)
  pltpu.sync_copy(zl_zero, acc)
  pltpu.sync_copy(src1.at[pl.ds(wid * EPT, EPT)], sidx)
  pltpu.sync_copy(dst1.at[pl.ds(wid * EPT, EPT)], didx)

  def body(j, carry):
    sv = sidx[pl.ds(16 * j, 16)]
    dv = didx[pl.ds(16 * j, 16)]
    s2 = sv * 2
    d2 = dv * 2
    v0 = plsc.load_gather(tab, [s2])
    v1 = plsc.load_gather(tab, [s2 + 1])
    plsc.addupdate_scatter(acc, [d2], v0)
    plsc.addupdate_scatter(acc, [d2 + 1], v1)
    return carry

  lax.fori_loop(0, EPT // 16, body, 0)

  pltpu.sync_copy(acc, stage.at[s])
  plsc.subcore_barrier()
  pltpu.sync_copy(stage.at[0, pl.ds(s * CSL, CSL)], csum)
  for t in range(1, NS):
    pltpu.sync_copy(stage.at[t, pl.ds(s * CSL, CSL)], cin)

    def addb(j, carry):
      csum[pl.ds(16 * j, 16)] = (csum[pl.ds(16 * j, 16)]
                                 + cin[pl.ds(16 * j, 16)])
      return carry

    lax.fori_loop(0, CSL // 16, addb, 0)
  pltpu.sync_copy(csum, out.at[pl.ds(c * ZL + s * CSL, CSL)])


@functools.partial(
    pl.kernel,
    out_type=jax.ShapeDtypeStruct((NC * NPAD,), jnp.float32),
    mesh=_mesh,
    compiler_params=_sc_params,
    scratch_types=[
        pltpu.VMEM_SHARED((NPAD,), jnp.float32),
        pltpu.VMEM((EPT,), jnp.int32),
        pltpu.VMEM((EPT,), jnp.float32),
    ],
)
def _hist_kernel(dst1, zrows, ones_h, out, acc, didx, onesv):
  c = lax.axis_index("c")
  s = lax.axis_index("s")
  wid = c * NS + s
  pltpu.sync_copy(zrows, acc.at[pl.ds(s * RPT, RPT)])
  pltpu.sync_copy(dst1.at[pl.ds(wid * EPT, EPT)], didx)
  pltpu.sync_copy(ones_h, onesv)
  plsc.subcore_barrier()
  pltpu.sync_copy(onesv, acc.at[didx], add=True)
  plsc.subcore_barrier()
  pltpu.sync_copy(acc.at[pl.ds(s * RPT, RPT)],
                  out.at[pl.ds(c * NPAD + s * RPT, RPT)])


def _mm_body(x_ref, w1_ref, h_ref):
  h_ref[...] = jnp.dot(x_ref[...], w1_ref[...],
                       preferred_element_type=jnp.float32)


def _dense1_body(hp_ref, h_ref, hs_ref, dis_ref):
  deg = hp_ref[0:NPAD] + hp_ref[NPAD:2 * NPAD] + 1.0
  dis = lax.rsqrt(deg)
  dis_ref[...] = dis
  hs_ref[0:N] = h_ref[...] * dis[0:N][:, None]


def _dense2_body(accs_ref, hs_ref, dis_ref, b1_ref, w2_ref, zs_ref):
  acc = accs_ref[0:N] + accs_ref[NPAD:NPAD + N]
  dis = dis_ref[0:N][:, None]
  out1 = jnp.maximum(dis * (acc + hs_ref[0:N]) + b1_ref[...], 0.0)
  z = jnp.dot(out1, w2_ref[...], preferred_element_type=jnp.float32)
  zs_ref[0:N] = (dis * z)[:, 0:2]


def _final_body(acc2_ref, zf_ref, di_ref, b2i_ref, out_ref):
  a = acc2_ref[0:2 * NGEN] + acc2_ref[2 * NPAD:2 * NPAD + 2 * NGEN]
  out_ref[...] = di_ref[...] * (a + zf_ref[0:2 * NGEN]) + b2i_ref[...]


def kernel(x, edge_index, W1, b1, W2, b2):
  src = edge_index[0].astype(jnp.int32)
  dst = edge_index[1].astype(jnp.int32)
  padi = N + jnp.arange(EPAD - E, dtype=jnp.int32) % (NPAD - N)
  srcp = jnp.concatenate([src, padi])
  dstp = jnp.concatenate([dst, padi])

  z64 = jnp.zeros((RPT, H), jnp.float32)
  z1 = jnp.zeros((RPT,), jnp.float32)
  z2 = jnp.zeros((ZL,), jnp.float32)
  ones1 = jnp.ones((EPT,), jnp.float32)

  h = pl.pallas_call(
      _mm_body, out_shape=jax.ShapeDtypeStruct((N, H), jnp.float32),
  )(x, W1)

  hp1 = _hist_kernel(dstp, z1, ones1)

  hs, dis1 = pl.pallas_call(
      _dense1_body,
      out_shape=(jax.ShapeDtypeStruct((NPAD, H), jnp.float32),
                 jax.ShapeDtypeStruct((NPAD,), jnp.float32)),
  )(hp1, h)

  accs = _edge_pass64(hs, srcp, dstp, z64)

  W2p = jnp.zeros((H, 8), jnp.float32).at[:, 0:2].set(W2)
  zs = pl.pallas_call(
      _dense2_body,
      out_shape=jax.ShapeDtypeStruct((NPAD, 2), jnp.float32),
  )(accs, hs, dis1, b1, W2p)

  zsf = zs.reshape(-1)
  acc2f = _edge_pass2(zsf, srcp, dstp, z2)

  di = jnp.repeat(dis1[0:NGEN], 2)
  b2i = jnp.tile(b2, NGEN)
  out = pl.pallas_call(
      _final_body,
      out_shape=jax.ShapeDtypeStruct((2 * NGEN,), jnp.float32),
  )(acc2f, zsf, di, b2i)

  return out


# speedup vs baseline: 3.6863x; 1.0292x over previous
import functools

import jax
import jax.numpy as jnp
from jax import lax
from jax.experimental import pallas as pl
from jax.experimental.pallas import tpu as pltpu
from jax.experimental.pallas import tpu_sc as plsc

N = 10000
D = 128
H = 64
E = 320000
NGEN = 1024

NC = 2
NS = 16
NW = NC * NS
NPAD = 10112
RPT = NPAD // NS
EPAD = 327680
EPT = EPAD // NW
CH = 128
K = 4
GRP = K * CH
NG = EPT // GRP

_mesh = plsc.VectorSubcoreMesh(core_axis_name="c", subcore_axis_name="s")
_sc_params = pltpu.CompilerParams(use_tc_tiling_on_sc=False)


@functools.partial(
    pl.kernel,
    out_type=jax.ShapeDtypeStruct((NC * NPAD, H), jnp.float32),
    mesh=_mesh,
    compiler_params=_sc_params,
    scratch_types=[
        pltpu.VMEM_SHARED((NPAD, H), jnp.float32),
        pltpu.VMEM((EPT,), jnp.int32),
        pltpu.VMEM((EPT,), jnp.int32),
        pltpu.VMEM((2 * K, CH, H), jnp.float32),
        pltpu.SemaphoreType.DMA,
        pltpu.SemaphoreType.DMA,
    ],
)
def _edge_pass64(table, src1, dst1, zrows, out, acc, sidx, didx, rbuf,
                 sem0, sem1):
  c = lax.axis_index("c")
  s = lax.axis_index("s")
  wid = c * NS + s
  pltpu.sync_copy(zrows, acc.at[pl.ds(s * RPT, RPT)])
  pltpu.sync_copy(src1.at[pl.ds(wid * EPT, EPT)], sidx)
  pltpu.sync_copy(dst1.at[pl.ds(wid * EPT, EPT)], didx)
  plsc.subcore_barrier()

  sems = (sem0, sem1)

  def fire(g, b):
    for i in range(K):
      pltpu.async_copy(table.at[sidx.at[pl.ds(g * GRP + i * CH, CH)]],
                       rbuf.at[b * K + i], sems[b])

  def wait_scatter(g, b):
    for i in range(K):
      pltpu.make_async_copy(table.at[sidx.at[pl.ds(g * GRP + i * CH, CH)]],
                            rbuf.at[b * K + i], sems[b]).wait()
      pltpu.sync_copy(rbuf.at[b * K + i],
                      acc.at[didx.at[pl.ds(g * GRP + i * CH, CH)]], add=True)

  fire(0, 0)
  fire(1, 1)

  def body(p, carry):
    g0 = 2 * p
    wait_scatter(g0, 0)
    fire(g0 + 2, 0)
    wait_scatter(g0 + 1, 1)
    fire(g0 + 3, 1)
    return carry

  lax.fori_loop(0, NG // 2 - 1, body, 0)
  wait_scatter(NG - 2, 0)
  wait_scatter(NG - 1, 1)

  plsc.subcore_barrier()
  pltpu.sync_copy(acc.at[pl.ds(s * RPT, RPT)],
                  out.at[pl.ds(c * NPAD + s * RPT, RPT)])


ZL = 2 * NPAD
CSL = ZL // NS


@functools.partial(
    pl.kernel,
    out_type=jax.ShapeDtypeStruct((NC * ZL,), jnp.float32),
    mesh=_mesh,
    compiler_params=pltpu.CompilerParams(use_tc_tiling_on_sc=False,
                                         needs_layout_passes=False),
    scratch_types=[
        pltpu.VMEM_SHARED((NS, ZL), jnp.float32),
        pltpu.VMEM((ZL,), jnp.float32),
        pltpu.VMEM((ZL,), jnp.float32),
        pltpu.VMEM((EPT,), jnp.int32),
        pltpu.VMEM((EPT,), jnp.int32),
        pltpu.VMEM((CSL,), jnp.float32),
        pltpu.VMEM((CSL,), jnp.float32),
    ],
)
def _edge_pass2(zsf, src1, dst1, zl_zero, out, stage, tab, acc, sidx, didx,
                cin, csum):
  c = lax.axis_index("c")
  s = lax.axis_index("s")
  wid = c * NS + s
  pltpu.sync_copy(zsf, tab)
  pltpu.sync_copy(zl_zero, acc)
  pltpu.sync_copy(src1.at[pl.ds(wid * EPT, EPT)], sidx)
  pltpu.sync_copy(dst1.at[pl.ds(wid * EPT, EPT)], didx)

  def body(j, carry):
    for u in range(8):
      sv = sidx[pl.ds(128 * j + 16 * u, 16)]
      dv = didx[pl.ds(128 * j + 16 * u, 16)]
      s2 = sv * 2
      d2 = dv * 2
      v0 = plsc.load_gather(tab, [s2])
      v1 = plsc.load_gather(tab, [s2 + 1])
      plsc.addupdate_scatter(acc, [d2], v0)
      plsc.addupdate_scatter(acc, [d2 + 1], v1)
    return carry

  lax.fori_loop(0, EPT // 128, body, 0)

  pltpu.sync_copy(acc, stage.at[s])
  plsc.subcore_barrier()
  pltpu.sync_copy(stage.at[0, pl.ds(s * CSL, CSL)], csum)
  for t in range(1, NS):
    pltpu.sync_copy(stage.at[t, pl.ds(s * CSL, CSL)], cin)

    def addb(j, carry):
      for u in range(4):
        o = 64 * j + 16 * u
        csum[pl.ds(o, 16)] = csum[pl.ds(o, 16)] + cin[pl.ds(o, 16)]
      return carry

    lax.fori_loop(0, CSL // 64, addb, 0)
    for o in range(CSL - CSL % 64, CSL, 16):
      csum[pl.ds(o, 16)] = csum[pl.ds(o, 16)] + cin[pl.ds(o, 16)]
  pltpu.sync_copy(csum, out.at[pl.ds(c * ZL + s * CSL, CSL)])


@functools.partial(
    pl.kernel,
    out_type=jax.ShapeDtypeStruct((NC * NPAD,), jnp.float32),
    mesh=_mesh,
    compiler_params=_sc_params,
    scratch_types=[
        pltpu.VMEM_SHARED((NPAD,), jnp.float32),
        pltpu.VMEM((EPT,), jnp.int32),
        pltpu.VMEM((EPT,), jnp.float32),
    ],
)
def _hist_kernel(dst1, zrows, ones_h, out, acc, didx, onesv):
  c = lax.axis_index("c")
  s = lax.axis_index("s")
  wid = c * NS + s
  pltpu.sync_copy(zrows, acc.at[pl.ds(s * RPT, RPT)])
  pltpu.sync_copy(dst1.at[pl.ds(wid * EPT, EPT)], didx)
  pltpu.sync_copy(ones_h, onesv)
  plsc.subcore_barrier()
  pltpu.sync_copy(onesv, acc.at[didx], add=True)
  plsc.subcore_barrier()
  pltpu.sync_copy(acc.at[pl.ds(s * RPT, RPT)],
                  out.at[pl.ds(c * NPAD + s * RPT, RPT)])


def _mm_body(x_ref, w1_ref, h_ref):
  h_ref[...] = jnp.dot(x_ref[...], w1_ref[...],
                       preferred_element_type=jnp.float32)


def _dense1_body(hp_ref, h_ref, hs_ref, dis_ref):
  deg = hp_ref[0:NPAD] + hp_ref[NPAD:2 * NPAD] + 1.0
  dis = lax.rsqrt(deg)
  dis_ref[...] = dis
  hs_ref[0:N] = h_ref[...] * dis[0:N][:, None]


def _dense2_body(accs_ref, hs_ref, dis_ref, b1_ref, w2_ref, zs_ref):
  acc = accs_ref[0:N] + accs_ref[NPAD:NPAD + N]
  dis = dis_ref[0:N][:, None]
  out1 = jnp.maximum(dis * (acc + hs_ref[0:N]) + b1_ref[...], 0.0)
  z = jnp.dot(out1, w2_ref[...], preferred_element_type=jnp.float32)
  zs_ref[0:N] = (dis * z)[:, 0:2]


def _final_body(acc2_ref, zf_ref, di_ref, b2i_ref, out_ref):
  a = acc2_ref[0:2 * NGEN] + acc2_ref[2 * NPAD:2 * NPAD + 2 * NGEN]
  out_ref[...] = di_ref[...] * (a + zf_ref[0:2 * NGEN]) + b2i_ref[...]


def kernel(x, edge_index, W1, b1, W2, b2):
  src = edge_index[0].astype(jnp.int32)
  dst = edge_index[1].astype(jnp.int32)
  padi = N + jnp.arange(EPAD - E, dtype=jnp.int32) % (NPAD - N)
  srcp = jnp.concatenate([src, padi])
  dstp = jnp.concatenate([dst, padi])

  z64 = jnp.zeros((RPT, H), jnp.float32)
  z1 = jnp.zeros((RPT,), jnp.float32)
  z2 = jnp.zeros((ZL,), jnp.float32)
  ones1 = jnp.ones((EPT,), jnp.float32)

  h = pl.pallas_call(
      _mm_body, out_shape=jax.ShapeDtypeStruct((N, H), jnp.float32),
  )(x, W1)

  hp1 = _hist_kernel(dstp, z1, ones1)

  hs, dis1 = pl.pallas_call(
      _dense1_body,
      out_shape=(jax.ShapeDtypeStruct((NPAD, H), jnp.float32),
                 jax.ShapeDtypeStruct((NPAD,), jnp.float32)),
  )(hp1, h)

  accs = _edge_pass64(hs, srcp, dstp, z64)

  W2p = jnp.zeros((H, 8), jnp.float32).at[:, 0:2].set(W2)
  zs = pl.pallas_call(
      _dense2_body,
      out_shape=jax.ShapeDtypeStruct((NPAD, 2), jnp.float32),
  )(accs, hs, dis1, b1, W2p)

  zsf = zs.reshape(-1)
  acc2f = _edge_pass2(zsf, srcp, dstp, z2)

  di = jnp.repeat(dis1[0:NGEN], 2)
  b2i = jnp.tile(b2, NGEN)
  out = pl.pallas_call(
      _final_body,
      out_shape=jax.ShapeDtypeStruct((2 * NGEN,), jnp.float32),
  )(acc2f, zsf, di, b2i)

  return out
